# baseline (speedup 1.0000x reference)
import functools

import jax
import jax.numpy as jnp
from jax import lax
from jax.experimental import pallas as pl
from jax.experimental.pallas import tpu as pltpu
from jax.experimental.pallas import tpu_sc as plsc

VOCAB = 100000
DIM = 128
CTX = 20
LATENT = 128

RBLK = 5000
NBLK = VOCAB // RBLK


def _sc_gather(idx, table):
    mesh = plsc.VectorSubcoreMesh(core_axis_name="c", subcore_axis_name="s")

    @functools.partial(
        pl.kernel,
        mesh=mesh,
        out_type=jax.ShapeDtypeStruct((CTX, DIM), jnp.float32),
        scratch_types=[
            pltpu.VMEM((CTX,), jnp.int32),
            pltpu.VMEM((CTX, DIM), jnp.float32),
            pltpu.SemaphoreType.DMA,
        ],
    )
    def gather_k(idx_hbm, table_hbm, out_hbm, idx_v, rows_v, sem):
        wid = lax.axis_index("s") * 2 + lax.axis_index("c")

        @pl.when(wid == 0)
        def _():
            pltpu.sync_copy(idx_hbm, idx_v)
            pltpu.async_copy(table_hbm.at[idx_v], rows_v, sem).wait()
            pltpu.sync_copy(rows_v, out_hbm)

    return gather_k(idx, table)


def _logits_kernel(x_ref, w1_ref, b1_ref, w2_ref, b2_ref,
                   out_ref, h_scr, logit_scr, acc_scr):
    p = pl.program_id(0)
    b = pl.program_id(1)

    @pl.when((p == 0) & (b == 0))
    def _():
        h = lax.dot_general(x_ref[...], w1_ref[...],
                            (((1,), (1,)), ((), ())),
                            preferred_element_type=jnp.float32)
        h_scr[0:1, :] = jnp.maximum(h + b1_ref[...], 0.0)
        acc_scr[0] = -jnp.inf
        acc_scr[1] = 0.0

    @pl.when(p == 0)
    def _():
        h = h_scr[0:1, :]
        w2 = w2_ref[0]
        logits = lax.dot_general(h, w2, (((1,), (1,)), ((), ())),
                                 preferred_element_type=jnp.float32)
        logits = logits + b2_ref[0]
        logit_scr[b] = logits

        m_old = acc_scr[0]
        m_new = jnp.maximum(m_old, jnp.max(logits))
        s_new = (acc_scr[1] * jnp.exp(m_old - m_new)
                 + jnp.sum(jnp.exp(logits - m_new)))
        acc_scr[0] = m_new
        acc_scr[1] = s_new

        @pl.when(b == NBLK - 1)
        def _():
            acc_scr[2] = m_new + jnp.log(s_new)

    @pl.when(p == 1)
    def _():
        out_ref[0] = logit_scr[b] - acc_scr[2]


def kernel(inputs, table, W1, b1, W2, b2):
    idx = inputs.astype(jnp.int32)
    embeds = _sc_gather(idx, table).reshape(1, CTX * DIM)

    w2_blocks = W2.reshape(NBLK, RBLK, DIM)
    b2_blocks = b2.reshape(NBLK, 1, RBLK)

    log_probs = pl.pallas_call(
        _logits_kernel,
        grid=(2, NBLK),
        in_specs=[
            pl.BlockSpec((1, CTX * DIM), lambda p, b: (0, 0)),
            pl.BlockSpec((LATENT, CTX * DIM), lambda p, b: (0, 0)),
            pl.BlockSpec((1, LATENT), lambda p, b: (0, 0)),
            pl.BlockSpec((1, RBLK, DIM),
                         lambda p, b: (jnp.where(p == 0, b, NBLK - 1), 0, 0)),
            pl.BlockSpec((1, 1, RBLK),
                         lambda p, b: (jnp.where(p == 0, b, NBLK - 1), 0, 0)),
        ],
        out_specs=pl.BlockSpec(
            (1, 1, RBLK), lambda p, b: (jnp.where(p == 0, 0, b), 0, 0)),
        out_shape=jax.ShapeDtypeStruct((NBLK, 1, RBLK), jnp.float32),
        scratch_shapes=[
            pltpu.VMEM((8, LATENT), jnp.float32),
            pltpu.VMEM((NBLK, 1, RBLK), jnp.float32),
            pltpu.SMEM((3,), jnp.float32),
        ],
        compiler_params=pltpu.CompilerParams(
            dimension_semantics=("arbitrary", "arbitrary")),
    )(embeds, W1, b1.reshape(1, LATENT), w2_blocks, b2_blocks)

    return log_probs.reshape(1, VOCAB)


# speedup vs baseline: 1.1868x; 1.1868x over previous
import functools

import jax
import jax.numpy as jnp
from jax import lax
from jax.experimental import pallas as pl
from jax.experimental.pallas import tpu as pltpu
from jax.experimental.pallas import tpu_sc as plsc

VOCAB = 100000
DIM = 128
CTX = 20
LATENT = 128

RBLK = 10000
NBLK = VOCAB // RBLK


def _sc_gather(idx, table):
    mesh = plsc.VectorSubcoreMesh(core_axis_name="c", subcore_axis_name="s")

    @functools.partial(
        pl.kernel,
        mesh=mesh,
        out_type=jax.ShapeDtypeStruct((CTX, DIM), jnp.float32),
        scratch_types=[
            pltpu.VMEM((CTX,), jnp.int32),
            pltpu.VMEM((CTX, DIM), jnp.float32),
            pltpu.SemaphoreType.DMA,
        ],
    )
    def gather_k(idx_hbm, table_hbm, out_hbm, idx_v, rows_v, sem):
        wid = lax.axis_index("s") * 2 + lax.axis_index("c")

        @pl.when(wid == 0)
        def _():
            pltpu.sync_copy(idx_hbm, idx_v)
            pltpu.async_copy(table_hbm.at[idx_v], rows_v, sem).wait()
            pltpu.sync_copy(rows_v, out_hbm)

    return gather_k(idx, table)


def _logits_kernel(x_ref, w1_ref, b1_ref, w2_ref, b2_ref,
                   logits_ref, lse_ref, h_scr, acc_scr):
    b = pl.program_id(0)

    @pl.when(b == 0)
    def _():
        h = lax.dot_general(x_ref[...], w1_ref[...],
                            (((1,), (1,)), ((), ())),
                            preferred_element_type=jnp.float32)
        h_scr[0:1, :] = jnp.maximum(h + b1_ref[...], 0.0)
        acc_scr[0] = -jnp.inf
        acc_scr[1] = 0.0

    h = h_scr[0:1, :]
    w2 = w2_ref[0]
    logits = lax.dot_general(h, w2, (((1,), (1,)), ((), ())),
                             preferred_element_type=jnp.float32)
    logits = logits + b2_ref[0]
    logits_ref[0] = logits

    m_old = acc_scr[0]
    m_new = jnp.maximum(m_old, jnp.max(logits))
    s_new = (acc_scr[1] * jnp.exp(m_old - m_new)
             + jnp.sum(jnp.exp(logits - m_new)))
    acc_scr[0] = m_new
    acc_scr[1] = s_new

    @pl.when(b == NBLK - 1)
    def _():
        lse_ref[0, 0] = m_new + jnp.log(s_new)


def _norm_kernel(logits_ref, lse_ref, out_ref):
    out_ref[...] = logits_ref[...] - lse_ref[0, 0]


def kernel(inputs, table, W1, b1, W2, b2):
    idx = inputs.astype(jnp.int32)
    embeds = _sc_gather(idx, table).reshape(1, CTX * DIM)

    w2_blocks = W2.reshape(NBLK, RBLK, DIM)
    b2_blocks = b2.reshape(NBLK, 1, RBLK)

    logits, lse = pl.pallas_call(
        _logits_kernel,
        grid=(NBLK,),
        in_specs=[
            pl.BlockSpec((1, CTX * DIM), lambda b: (0, 0)),
            pl.BlockSpec((LATENT, CTX * DIM), lambda b: (0, 0)),
            pl.BlockSpec((1, LATENT), lambda b: (0, 0)),
            pl.BlockSpec((1, RBLK, DIM), lambda b: (b, 0, 0)),
            pl.BlockSpec((1, 1, RBLK), lambda b: (b, 0, 0)),
        ],
        out_specs=[
            pl.BlockSpec((1, 1, RBLK), lambda b: (b, 0, 0)),
            pl.BlockSpec(memory_space=pltpu.SMEM),
        ],
        out_shape=[
            jax.ShapeDtypeStruct((NBLK, 1, RBLK), jnp.float32),
            jax.ShapeDtypeStruct((1, 1), jnp.float32),
        ],
        scratch_shapes=[
            pltpu.VMEM((8, LATENT), jnp.float32),
            pltpu.SMEM((2,), jnp.float32),
        ],
        compiler_params=pltpu.CompilerParams(
            dimension_semantics=("arbitrary",)),
    )(embeds, W1, b1.reshape(1, LATENT), w2_blocks, b2_blocks)

    log_probs = pl.pallas_call(
        _norm_kernel,
        in_specs=[
            pl.BlockSpec((NBLK, 1, RBLK), lambda: (0, 0, 0)),
            pl.BlockSpec(memory_space=pltpu.SMEM),
        ],
        out_specs=pl.BlockSpec((NBLK, 1, RBLK), lambda: (0, 0, 0)),
        out_shape=jax.ShapeDtypeStruct((NBLK, 1, RBLK), jnp.float32),
    )(logits, lse)

    return log_probs.reshape(1, VOCAB)
